# baseline (speedup 1.0000x reference)
import functools

import jax
import jax.numpy as jnp
from jax import lax
from jax.experimental import pallas as pl
from jax.experimental.pallas import tpu as pltpu
from jax.experimental.pallas import tpu_sc as plsc

B = 4096
L = 200
D = 64
NB = B * L
N_TOK = 1000000
SCALE = 8.0

_INFO = plsc.get_sparse_core_info()
NC = _INFO.num_cores
NS = _INFO.num_subcores
NW = NC * NS
BPW = NB // NW
C = 256
NCHUNK = BPW // C

VBLK = 512
NBLK = pl.cdiv(N_TOK, VBLK)

_mesh = plsc.VectorSubcoreMesh(core_axis_name="c", subcore_axis_name="s")


def _transpose_blk(tt_ref, out_ref):
    blk = tt_ref[...]
    eye = jnp.eye(D, dtype=jnp.float32) * SCALE
    t = lax.dot_general(blk, eye, (((0,), (0,)), ((), ())),
                        precision=lax.Precision.HIGHEST)
    out_ref[:, 0:D] = t
    out_ref[:, D : 2 * D] = jnp.zeros((VBLK, D), jnp.float32)


_prep = pl.pallas_call(
    _transpose_blk,
    grid=(NBLK,),
    in_specs=[pl.BlockSpec((D, VBLK), lambda j: (0, j))],
    out_specs=pl.BlockSpec((VBLK, 2 * D), lambda j: (j, 0)),
    out_shape=jax.ShapeDtypeStruct((N_TOK, 2 * D), jnp.float32),
    compiler_params=pltpu.CompilerParams(
        dimension_semantics=("arbitrary",)),
)


@functools.partial(
    pl.kernel,
    mesh=_mesh,
    compiler_params=pltpu.CompilerParams(use_tc_tiling_on_sc=True),
    out_type=jax.ShapeDtypeStruct((NB, D), jnp.float32),
    scratch_types=[
        pltpu.VMEM((C,), jnp.int32),
        pltpu.VMEM((C,), jnp.int32),
        pltpu.VMEM((C, 2 * D), jnp.float32),
        pltpu.VMEM((C, 2 * D), jnp.float32),
        pltpu.VMEM((C, D), jnp.float32),
        pltpu.SemaphoreType.DMA,
        pltpu.SemaphoreType.DMA,
    ],
)
def _emb(idx_hbm, tw_hbm, out_hbm,
         idx0, idx1, wide0, wide1, st, sem0, sem1):
    wid = lax.axis_index("s") * NC + lax.axis_index("c")
    base = wid * BPW
    idx_v = (idx0, idx1)
    wide_v = (wide0, wide1)
    sems = (sem0, sem1)

    def issue(g, b):
        off = base + g * C
        pltpu.sync_copy(idx_hbm.at[pl.ds(off, C)], idx_v[b])
        pltpu.async_copy(tw_hbm.at[idx_v[b]], wide_v[b], sems[b])

    def drain_and_flush(g, b):
        pltpu.make_async_copy(tw_hbm.at[idx_v[b]], wide_v[b], sems[b]).wait()

        def row(t, c):
            for j in range(D // 16):
                sl = pl.ds(j * 16, 16)
                st[t, sl] = wide_v[b][t, sl]
            return c

        lax.fori_loop(0, C, row, 0, unroll=4)
        pltpu.sync_copy(st, out_hbm.at[pl.ds(base + g * C, C)])

        @pl.when(g + 2 < NCHUNK)
        def _():
            issue(g + 2, b)

    issue(0, 0)
    issue(1, 1)

    def pair(i, carry):
        g = i * 2
        drain_and_flush(g, 0)
        drain_and_flush(g + 1, 1)
        return carry

    lax.fori_loop(0, NCHUNK // 2, pair, 0)


def kernel(x, table):
    idx = x.reshape(NB).astype(jnp.int32)
    tw = _prep(table.T)
    out = _emb(idx, tw)
    return out.reshape(B, L, D)


# speedup vs baseline: 2.0864x; 2.0864x over previous
import functools

import jax
import jax.numpy as jnp
from jax import lax
from jax.experimental import pallas as pl
from jax.experimental.pallas import tpu as pltpu
from jax.experimental.pallas import tpu_sc as plsc

B = 4096
L = 200
D = 64
NB = B * L
N_TOK = 1000000
SCALE = 8.0

_INFO = plsc.get_sparse_core_info()
NC = _INFO.num_cores
NS = _INFO.num_subcores
NW = NC * NS
BPW = NB // NW
C = 128
NCHUNK = BPW // C
NBUF = 4
NST = 2

_mesh = plsc.VectorSubcoreMesh(core_axis_name="c", subcore_axis_name="s")


@functools.partial(
    pl.kernel,
    mesh=_mesh,
    compiler_params=pltpu.CompilerParams(use_tc_tiling_on_sc=True),
    out_type=jax.ShapeDtypeStruct((NB, D), jnp.float32),
    scratch_types=[
        pltpu.VMEM((NBUF, C), jnp.int32),
        pltpu.VMEM((C, 2 * D), jnp.float32),
        pltpu.VMEM((C, 2 * D), jnp.float32),
        pltpu.VMEM((C, 2 * D), jnp.float32),
        pltpu.VMEM((C, 2 * D), jnp.float32),
        pltpu.VMEM((C, D), jnp.float32),
        pltpu.VMEM((C, D), jnp.float32),
        pltpu.SemaphoreType.DMA,
        pltpu.SemaphoreType.DMA,
        pltpu.SemaphoreType.DMA,
        pltpu.SemaphoreType.DMA,
        pltpu.SemaphoreType.DMA,
        pltpu.SemaphoreType.DMA,
    ],
)
def _emb(idx_hbm, tw_hbm, out_hbm, idx_r, w0, w1, w2, w3, st0, st1,
         sg0, sg1, sg2, sg3, sw0, sw1):
    wid = lax.axis_index("s") * NC + lax.axis_index("c")
    base = wid * BPW
    wide_v = (w0, w1, w2, w3)
    st_v = (st0, st1)
    sg = (sg0, sg1, sg2, sg3)
    sw = (sw0, sw1)

    def issue(g, b):
        off = base + g * C
        pltpu.sync_copy(idx_hbm.at[pl.ds(off, C)], idx_r.at[b])
        pltpu.async_copy(tw_hbm.at[idx_r.at[b]], wide_v[b], sg[b])

    def wait_gather(b):
        pltpu.make_async_copy(tw_hbm.at[idx_r.at[b]], wide_v[b], sg[b]).wait()

    def wait_wout(sb, g):
        pltpu.make_async_copy(
            st_v[sb], out_hbm.at[pl.ds(base + g * C, C)], sw[sb]).wait()

    def step(g, k, i):
        b = k % NBUF
        sb = k % NST
        wait_gather(b)
        if k < NST:
            @pl.when(i > 0)
            def _():
                wait_wout(sb, g)
        else:
            wait_wout(sb, g)
        for t in range(C):
            for j in range(D // 16):
                sl = pl.ds(j * 16, 16)
                st_v[sb][t, sl] = wide_v[b][t, sl] * SCALE
        pltpu.async_copy(
            st_v[sb], out_hbm.at[pl.ds(base + g * C, C)], sw[sb])

        @pl.when(g + NBUF < NCHUNK)
        def _():
            issue(g + NBUF, b)

    for b in range(NBUF):
        issue(b, b)

    def block(i, carry):
        g0 = i * NBUF
        for k in range(NBUF):
            step(g0 + k, k, i)
        return carry

    lax.fori_loop(0, NCHUNK // NBUF, block, 0)
    wait_wout(0, 0)
    wait_wout(1, 1)


def kernel(x, table):
    idx = x.reshape(NB).astype(jnp.int32)
    tw = jnp.pad(table, ((0, 0), (0, D)))
    out = _emb(idx, tw)
    return out.reshape(B, L, D)


# speedup vs baseline: 2.1158x; 1.0141x over previous
import functools

import jax
import jax.numpy as jnp
from jax import lax
from jax.experimental import pallas as pl
from jax.experimental.pallas import tpu as pltpu
from jax.experimental.pallas import tpu_sc as plsc

B = 4096
L = 200
D = 64
NB = B * L
N_TOK = 1000000
SCALE = 8.0

_INFO = plsc.get_sparse_core_info()
NC = _INFO.num_cores
NS = _INFO.num_subcores
NW = NC * NS
BPW = NB // NW
C = 128
NCHUNK = BPW // C
NBUF = 4
NST = 2

_mesh = plsc.VectorSubcoreMesh(core_axis_name="c", subcore_axis_name="s")


@functools.partial(
    pl.kernel,
    mesh=_mesh,
    compiler_params=pltpu.CompilerParams(use_tc_tiling_on_sc=True),
    out_type=jax.ShapeDtypeStruct((NB, D), jnp.float32),
    scratch_types=[
        pltpu.VMEM((NBUF, C), jnp.int32),
        pltpu.VMEM((C, 2 * D), jnp.float32),
        pltpu.VMEM((C, 2 * D), jnp.float32),
        pltpu.VMEM((C, 2 * D), jnp.float32),
        pltpu.VMEM((C, 2 * D), jnp.float32),
        pltpu.VMEM((C, D), jnp.float32),
        pltpu.VMEM((C, D), jnp.float32),
        pltpu.SemaphoreType.DMA,
        pltpu.SemaphoreType.DMA,
        pltpu.SemaphoreType.DMA,
        pltpu.SemaphoreType.DMA,
        pltpu.SemaphoreType.DMA,
        pltpu.SemaphoreType.DMA,
        pltpu.SemaphoreType.DMA,
        pltpu.SemaphoreType.DMA,
        pltpu.SemaphoreType.DMA,
        pltpu.SemaphoreType.DMA,
    ],
)
def _emb(idx_hbm, tw_hbm, out_hbm, idx_r, w0, w1, w2, w3, st0, st1,
         sg0, sg1, sg2, sg3, sw0, sw1, si0, si1, si2, si3):
    wid = lax.axis_index("s") * NC + lax.axis_index("c")
    base = wid * BPW
    wide_v = (w0, w1, w2, w3)
    st_v = (st0, st1)
    sg = (sg0, sg1, sg2, sg3)
    sw = (sw0, sw1)
    si = (si0, si1, si2, si3)

    def idx_copy(g, b):
        pltpu.async_copy(
            idx_hbm.at[pl.ds(base + g * C, C)], idx_r.at[b], si[b])

    def wait_idx(g, b):
        pltpu.make_async_copy(
            idx_hbm.at[pl.ds(base + g * C, C)], idx_r.at[b], si[b]).wait()

    def gather(b):
        pltpu.async_copy(tw_hbm.at[idx_r.at[b]], wide_v[b], sg[b])

    def wait_gather(b):
        pltpu.make_async_copy(tw_hbm.at[idx_r.at[b]], wide_v[b], sg[b]).wait()

    def wait_wout(sb, g):
        pltpu.make_async_copy(
            st_v[sb], out_hbm.at[pl.ds(base + g * C, C)], sw[sb]).wait()

    def step(g, k, i):
        b = k % NBUF
        sb = k % NST
        wait_gather(b)

        @pl.when(g + NBUF < NCHUNK)
        def _():
            idx_copy(g + NBUF, b)

        if k < NST:
            @pl.when(i > 0)
            def _():
                wait_wout(sb, g)
        else:
            wait_wout(sb, g)
        for t in range(C):
            for j in range(D // 16):
                sl = pl.ds(j * 16, 16)
                st_v[sb][t, sl] = wide_v[b][t, sl] * SCALE
        pltpu.async_copy(
            st_v[sb], out_hbm.at[pl.ds(base + g * C, C)], sw[sb])

        @pl.when(g + NBUF < NCHUNK)
        def _():
            wait_idx(g + NBUF, b)
            gather(b)

    for b in range(NBUF):
        pltpu.sync_copy(idx_hbm.at[pl.ds(base + b * C, C)], idx_r.at[b])
        gather(b)

    def block(i, carry):
        g0 = i * NBUF
        for k in range(NBUF):
            step(g0 + k, k, i)
        return carry

    lax.fori_loop(0, NCHUNK // NBUF, block, 0)
    wait_wout(0, 0)
    wait_wout(1, 1)


def kernel(x, table):
    idx = x.reshape(NB).astype(jnp.int32)
    tw = jnp.pad(table, ((0, 0), (0, D)))
    out = _emb(idx, tw)
    return out.reshape(B, L, D)
